# Initial kernel scaffold; baseline (speedup 1.0000x reference)
import functools

import jax
import jax.numpy as jnp
from jax import lax
from jax.experimental import pallas as pl
from jax.experimental.pallas import tpu as pltpu
from jax.experimental.pallas import tpu_sc as plsc

N = 10000
E = 160000
D = 256
DH = D // 2
NC = 2
NS = 16
CH = 128
EC = 1280
EPAD = EC * CH
NPAD = N + 16
RPT = NPAD // NS
RPO = N // NS

_mesh = plsc.VectorSubcoreMesh(core_axis_name="c", subcore_axis_name="s")


@functools.partial(
    pl.kernel,
    out_type=jax.ShapeDtypeStruct((NC, NPAD, 16), jnp.float32),
    mesh=_mesh,
    scratch_types=[
        pltpu.VMEM((EC // (NC * NS), CH), jnp.int32),
        pltpu.VMEM((CH, 16), jnp.float32),
        pltpu.VMEM_SHARED((NPAD, 16), jnp.float32),
    ],
)
def _sc_deg(col_hbm, zeros_hbm, ones_hbm, deg_out, idx_v, ones_v, acc_sh):
    c = lax.axis_index("c")
    s = lax.axis_index("s")
    npt = EC // (NC * NS)
    pltpu.sync_copy(col_hbm.at[pl.ds((c * NS + s) * npt, npt)], idx_v)
    pltpu.sync_copy(ones_hbm, ones_v)
    pltpu.sync_copy(zeros_hbm.at[pl.ds(s * RPT, RPT)],
                    acc_sh.at[pl.ds(s * RPT, RPT)])
    plsc.subcore_barrier()

    @pl.loop(0, npt)
    def _(j):
        pltpu.sync_copy(ones_v, acc_sh.at[idx_v.at[j]], add=True)

    plsc.subcore_barrier()
    pltpu.sync_copy(acc_sh.at[pl.ds(s * RPT, RPT)],
                    deg_out.at[c, pl.ds(s * RPT, RPT)])


_RB = 400


def _linear_body(x_ref, w_ref, degp_ref, h0_ref, h1_ref):
    dp = degp_ref[...]
    d = dp[0, :, 0] + dp[1, :, 0]
    dis = jnp.where(d > 0, lax.rsqrt(d), 0.0)
    h = lax.dot_general(x_ref[...], w_ref[...], (((1,), (1,)), ((), ())),
                        preferred_element_type=jnp.float32)
    hs = h * dis[:, None]
    h0_ref[...] = hs[:, :DH]
    h1_ref[...] = hs[:, DH:]


_tc_linear = pl.pallas_call(
    _linear_body,
    grid=(N // _RB,),
    in_specs=[
        pl.BlockSpec((_RB, D), lambda i: (i, 0)),
        pl.BlockSpec((D, D), lambda i: (0, 0)),
        pl.BlockSpec((NC, _RB, 16), lambda i: (0, i, 0)),
    ],
    out_specs=[
        pl.BlockSpec((_RB, DH), lambda i: (i, 0)),
        pl.BlockSpec((_RB, DH), lambda i: (i, 0)),
    ],
    out_shape=[
        jax.ShapeDtypeStruct((N, DH), jnp.float32),
        jax.ShapeDtypeStruct((N, DH), jnp.float32),
    ],
)


@functools.partial(
    pl.kernel,
    out_type=[
        jax.ShapeDtypeStruct((N, DH), jnp.float32),
        jax.ShapeDtypeStruct((N, DH), jnp.float32),
    ],
    mesh=_mesh,
    scratch_types=[
        pltpu.VMEM((EC // NS, CH), jnp.int32),
        pltpu.VMEM((EC // NS, CH), jnp.int32),
        pltpu.VMEM((CH, DH), jnp.float32),
        pltpu.VMEM_SHARED((NPAD, DH), jnp.float32),
    ],
)
def _sc_agg(row_hbm, col_hbm, h0_hbm, h1_hbm, zeros_hbm, agg0, agg1,
            ridx_v, cidx_v, gbuf, acc_sh):
    c = lax.axis_index("c")
    s = lax.axis_index("s")
    npt = EC // NS
    pltpu.sync_copy(row_hbm.at[pl.ds(s * npt, npt)], ridx_v)
    pltpu.sync_copy(col_hbm.at[pl.ds(s * npt, npt)], cidx_v)
    pltpu.sync_copy(zeros_hbm.at[pl.ds(s * RPT, RPT)],
                    acc_sh.at[pl.ds(s * RPT, RPT)])
    plsc.subcore_barrier()

    def half(h_hbm, agg_out):
        @pl.loop(0, npt)
        def _(j):
            pltpu.sync_copy(h_hbm.at[ridx_v.at[j]], gbuf)
            pltpu.sync_copy(gbuf, acc_sh.at[cidx_v.at[j]], add=True)

        plsc.subcore_barrier()
        pltpu.sync_copy(acc_sh.at[pl.ds(s * RPO, RPO)],
                        agg_out.at[pl.ds(s * RPO, RPO)])

    @pl.when(c == 0)
    def _():
        half(h0_hbm, agg0)

    @pl.when(c == 1)
    def _():
        half(h1_hbm, agg1)


def _out_body(a0_ref, a1_ref, degp_ref, b_ref, o_ref):
    dp = degp_ref[...]
    d = dp[0, :, 0] + dp[1, :, 0]
    dis = jnp.where(d > 0, lax.rsqrt(d), 0.0)
    agg = jnp.concatenate([a0_ref[...], a1_ref[...]], axis=1)
    o_ref[...] = agg * dis[:, None] + b_ref[...]


_tc_out = pl.pallas_call(
    _out_body,
    grid=(N // _RB,),
    in_specs=[
        pl.BlockSpec((_RB, DH), lambda i: (i, 0)),
        pl.BlockSpec((_RB, DH), lambda i: (i, 0)),
        pl.BlockSpec((NC, _RB, 16), lambda i: (0, i, 0)),
        pl.BlockSpec((1, D), lambda i: (0, 0)),
    ],
    out_specs=pl.BlockSpec((_RB, D), lambda i: (i, 0)),
    out_shape=jax.ShapeDtypeStruct((N, D), jnp.float32),
)


def kernel(x, edge_index, W, bias):
    row = edge_index[0]
    col = edge_index[1]
    pad = EPAD - E
    rowp = jnp.pad(row, (0, pad)).reshape(EC, CH)
    colp = jnp.pad(col, (0, pad), constant_values=N).reshape(EC, CH)

    z_deg = jnp.zeros((NPAD, 16), jnp.float32)
    ones_deg = jnp.ones((CH, 16), jnp.float32)
    z_agg = jnp.zeros((NPAD, DH), jnp.float32)

    degp = _sc_deg(colp, z_deg, ones_deg)
    degp_n = degp[:, :N, :]
    h0, h1 = _tc_linear(x, W, degp_n)
    agg0, agg1 = _sc_agg(rowp, colp, h0, h1, z_agg)
    return _tc_out(agg0, agg1, degp_n, bias.reshape(1, D))


# speedup vs baseline: 7.2842x; 7.2842x over previous
import functools

import jax
import jax.numpy as jnp
from jax import lax
from jax.experimental import pallas as pl
from jax.experimental.pallas import tpu as pltpu
from jax.experimental.pallas import tpu_sc as plsc

N = 10000
E = 160000
D = 256
DH = D // 2
NC = 2
NS = 16
CH = 128
EC = 1280
EPAD = EC * CH
NPAD = 10112
RPT = NPAD // NS

_mesh = plsc.VectorSubcoreMesh(core_axis_name="c", subcore_axis_name="s")


@functools.partial(
    pl.kernel,
    out_type=jax.ShapeDtypeStruct((NC, NPAD, 128), jnp.float32),
    mesh=_mesh,
    scratch_types=[
        pltpu.VMEM((EC // (NC * NS), CH), jnp.int32),
        pltpu.VMEM((CH, 128), jnp.float32),
        pltpu.VMEM_SHARED((NPAD, 128), jnp.float32),
    ],
)
def _sc_deg(col_hbm, zeros_hbm, ones_hbm, deg_out, idx_v, ones_v, acc_sh):
    c = lax.axis_index("c")
    s = lax.axis_index("s")
    npt = EC // (NC * NS)
    pltpu.sync_copy(col_hbm.at[pl.ds((c * NS + s) * npt, npt)], idx_v)
    pltpu.sync_copy(ones_hbm, ones_v)
    pltpu.sync_copy(zeros_hbm.at[pl.ds(s * RPT, RPT)],
                    acc_sh.at[pl.ds(s * RPT, RPT)])
    plsc.subcore_barrier()

    @pl.loop(0, npt)
    def _(j):
        pltpu.sync_copy(ones_v, acc_sh.at[idx_v.at[j]], add=True)

    plsc.subcore_barrier()
    pltpu.sync_copy(acc_sh.at[pl.ds(s * RPT, RPT)],
                    deg_out.at[c, pl.ds(s * RPT, RPT)])


_RB = 400


def _linear_body(x_ref, w_ref, degp_ref, h0_ref, h1_ref):
    dp = degp_ref[...]
    d = dp[0, :, 0] + dp[1, :, 0]
    dis = jnp.where(d > 0, lax.rsqrt(d), 0.0)
    h = lax.dot_general(x_ref[...], w_ref[...], (((1,), (1,)), ((), ())),
                        preferred_element_type=jnp.float32)
    hs = h * dis[:, None]
    h0_ref[...] = hs[:, :DH]
    h1_ref[...] = hs[:, DH:]


_tc_linear = pl.pallas_call(
    _linear_body,
    grid=(N // _RB,),
    in_specs=[
        pl.BlockSpec((_RB, D), lambda i: (i, 0)),
        pl.BlockSpec((D, D), lambda i: (0, 0)),
        pl.BlockSpec((NC, _RB, 128), lambda i: (0, i, 0)),
    ],
    out_specs=[
        pl.BlockSpec((_RB, DH), lambda i: (i, 0)),
        pl.BlockSpec((_RB, DH), lambda i: (i, 0)),
    ],
    out_shape=[
        jax.ShapeDtypeStruct((N, DH), jnp.float32),
        jax.ShapeDtypeStruct((N, DH), jnp.float32),
    ],
)


@functools.partial(
    pl.kernel,
    out_type=[
        jax.ShapeDtypeStruct((NPAD, DH), jnp.float32),
        jax.ShapeDtypeStruct((NPAD, DH), jnp.float32),
    ],
    mesh=_mesh,
    scratch_types=[
        pltpu.VMEM((EC // NS, CH), jnp.int32),
        pltpu.VMEM((EC // NS, CH), jnp.int32),
        pltpu.VMEM((CH, DH), jnp.float32),
        pltpu.VMEM_SHARED((NPAD, DH), jnp.float32),
    ],
)
def _sc_agg(row_hbm, col_hbm, h0_hbm, h1_hbm, zeros_hbm, agg0, agg1,
            ridx_v, cidx_v, gbuf, acc_sh):
    c = lax.axis_index("c")
    s = lax.axis_index("s")
    npt = EC // NS
    pltpu.sync_copy(row_hbm.at[pl.ds(s * npt, npt)], ridx_v)
    pltpu.sync_copy(col_hbm.at[pl.ds(s * npt, npt)], cidx_v)
    pltpu.sync_copy(zeros_hbm.at[pl.ds(s * RPT, RPT)],
                    acc_sh.at[pl.ds(s * RPT, RPT)])
    plsc.subcore_barrier()

    def half(h_hbm, agg_out):
        @pl.loop(0, npt)
        def _(j):
            pltpu.sync_copy(h_hbm.at[ridx_v.at[j]], gbuf)
            pltpu.sync_copy(gbuf, acc_sh.at[cidx_v.at[j]], add=True)

        plsc.subcore_barrier()
        pltpu.sync_copy(acc_sh.at[pl.ds(s * RPT, RPT)],
                        agg_out.at[pl.ds(s * RPT, RPT)])

    @pl.when(c == 0)
    def _():
        half(h0_hbm, agg0)

    @pl.when(c == 1)
    def _():
        half(h1_hbm, agg1)


def _out_body(a0_ref, a1_ref, degp_ref, b_ref, o_ref):
    dp = degp_ref[...]
    d = dp[0, :, 0] + dp[1, :, 0]
    dis = jnp.where(d > 0, lax.rsqrt(d), 0.0)
    agg = jnp.concatenate([a0_ref[...], a1_ref[...]], axis=1)
    o_ref[...] = agg * dis[:, None] + b_ref[...]


_tc_out = pl.pallas_call(
    _out_body,
    grid=(N // _RB,),
    in_specs=[
        pl.BlockSpec((_RB, DH), lambda i: (i, 0)),
        pl.BlockSpec((_RB, DH), lambda i: (i, 0)),
        pl.BlockSpec((NC, _RB, 128), lambda i: (0, i, 0)),
        pl.BlockSpec((1, D), lambda i: (0, 0)),
    ],
    out_specs=pl.BlockSpec((_RB, D), lambda i: (i, 0)),
    out_shape=jax.ShapeDtypeStruct((N, D), jnp.float32),
)


def kernel(x, edge_index, W, bias):
    row = edge_index[0]
    col = edge_index[1]
    pad = EPAD - E
    rowp = jnp.pad(row, (0, pad)).reshape(EC, CH)
    colp = jnp.pad(col, (0, pad), constant_values=N).reshape(EC, CH)

    z_deg = jnp.zeros((NPAD, 128), jnp.float32)
    ones_deg = jnp.ones((CH, 128), jnp.float32)
    z_agg = jnp.zeros((NPAD, DH), jnp.float32)

    degp = _sc_deg(colp, z_deg, ones_deg)
    degp_n = degp[:, :N, :]
    h0, h1 = _tc_linear(x, W, degp_n)
    agg0, agg1 = _sc_agg(rowp, colp, h0, h1, z_agg)
    return _tc_out(agg0[:N], agg1[:N], degp_n, bias.reshape(1, D))


# speedup vs baseline: 8.2504x; 1.1326x over previous
import functools

import jax
import jax.numpy as jnp
from jax import lax
from jax.experimental import pallas as pl
from jax.experimental.pallas import tpu as pltpu
from jax.experimental.pallas import tpu_sc as plsc

N = 10000
E = 160000
D = 256
DH = D // 2
NC = 2
NS = 16
CH = 128
EC = 1280
EPAD = EC * CH
NPAD = 10112
RPT = NPAD // NS
BLK = 16

_mesh = plsc.VectorSubcoreMesh(core_axis_name="c", subcore_axis_name="s")


@functools.partial(
    pl.kernel,
    out_type=jax.ShapeDtypeStruct((NC, NPAD, 128), jnp.float32),
    mesh=_mesh,
    scratch_types=[
        pltpu.VMEM((EC // (NC * NS), CH), jnp.int32),
        pltpu.VMEM((CH, 128), jnp.float32),
        pltpu.VMEM_SHARED((NPAD, 128), jnp.float32),
    ],
)
def _sc_deg(col_hbm, zeros_hbm, ones_hbm, deg_out, idx_v, ones_v, acc_sh):
    c = lax.axis_index("c")
    s = lax.axis_index("s")
    npt = EC // (NC * NS)
    pltpu.sync_copy(col_hbm.at[pl.ds((c * NS + s) * npt, npt)], idx_v)
    pltpu.sync_copy(ones_hbm, ones_v)
    pltpu.sync_copy(zeros_hbm.at[pl.ds(s * RPT, RPT)],
                    acc_sh.at[pl.ds(s * RPT, RPT)])
    plsc.subcore_barrier()

    @pl.loop(0, npt)
    def _(j):
        pltpu.sync_copy(ones_v, acc_sh.at[idx_v.at[j]], add=True)

    plsc.subcore_barrier()
    pltpu.sync_copy(acc_sh.at[pl.ds(s * RPT, RPT)],
                    deg_out.at[c, pl.ds(s * RPT, RPT)])


_RB = 400


def _linear_body(x_ref, w_ref, degp_ref, h0_ref, h1_ref):
    dp = degp_ref[...]
    d = dp[0, :, 0] + dp[1, :, 0]
    dis = jnp.where(d > 0, lax.rsqrt(d), 0.0)
    h = lax.dot_general(x_ref[...], w_ref[...], (((1,), (1,)), ((), ())),
                        preferred_element_type=jnp.float32)
    hs = h * dis[:, None]
    h0_ref[...] = hs[:, :DH]
    h1_ref[...] = hs[:, DH:]


_tc_linear = pl.pallas_call(
    _linear_body,
    grid=(N // _RB,),
    in_specs=[
        pl.BlockSpec((_RB, D), lambda i: (i, 0)),
        pl.BlockSpec((D, D), lambda i: (0, 0)),
        pl.BlockSpec((NC, _RB, 128), lambda i: (0, i, 0)),
    ],
    out_specs=[
        pl.BlockSpec((_RB, DH), lambda i: (i, 0)),
        pl.BlockSpec((_RB, DH), lambda i: (i, 0)),
    ],
    out_shape=[
        jax.ShapeDtypeStruct((N, DH), jnp.float32),
        jax.ShapeDtypeStruct((N, DH), jnp.float32),
    ],
)


@functools.partial(
    pl.kernel,
    out_type=[
        jax.ShapeDtypeStruct((NPAD, DH), jnp.float32),
        jax.ShapeDtypeStruct((NPAD, DH), jnp.float32),
    ],
    mesh=_mesh,
    scratch_types=[
        pltpu.VMEM((BLK, CH), jnp.int32),
        pltpu.VMEM((BLK, CH), jnp.int32),
        pltpu.VMEM((CH, DH), jnp.float32),
        pltpu.VMEM((CH, DH), jnp.float32),
        pltpu.SemaphoreType.DMA,
        pltpu.SemaphoreType.DMA,
        pltpu.VMEM_SHARED((NPAD, DH), jnp.float32),
    ],
)
def _sc_agg(row_hbm, col_hbm, h0_hbm, h1_hbm, zeros_hbm, agg0, agg1,
            ridx_v, cidx_v, gb0, gb1, sm0, sm1, acc_sh):
    c = lax.axis_index("c")
    s = lax.axis_index("s")
    npt = EC // NS
    gbufs = (gb0, gb1)
    sems = (sm0, sm1)
    pltpu.sync_copy(zeros_hbm.at[pl.ds(s * RPT, RPT)],
                    acc_sh.at[pl.ds(s * RPT, RPT)])
    plsc.subcore_barrier()

    def half(h_hbm, agg_out):
        @pl.loop(0, npt // BLK)
        def _(blk):
            base = s * npt + blk * BLK
            pltpu.sync_copy(row_hbm.at[pl.ds(base, BLK)], ridx_v)
            pltpu.sync_copy(col_hbm.at[pl.ds(base, BLK)], cidx_v)
            pltpu.async_copy(h_hbm.at[ridx_v.at[0]], gb0, sm0)

            @pl.loop(0, BLK, step=2)
            def _(j):
                for b in range(2):
                    jj = j + b

                    @pl.when(jj + 1 < BLK)
                    def _():
                        pltpu.async_copy(h_hbm.at[ridx_v.at[jj + 1]],
                                         gbufs[1 - b], sems[1 - b])

                    pltpu.make_async_copy(h_hbm.at[ridx_v.at[jj]],
                                          gbufs[b], sems[b]).wait()
                    pltpu.sync_copy(gbufs[b], acc_sh.at[cidx_v.at[jj]],
                                    add=True)

        plsc.subcore_barrier()
        pltpu.sync_copy(acc_sh.at[pl.ds(s * RPT, RPT)],
                        agg_out.at[pl.ds(s * RPT, RPT)])

    @pl.when(c == 0)
    def _():
        half(h0_hbm, agg0)

    @pl.when(c == 1)
    def _():
        half(h1_hbm, agg1)


def _out_body(a0_ref, a1_ref, degp_ref, b_ref, o_ref):
    dp = degp_ref[...]
    d = dp[0, :, 0] + dp[1, :, 0]
    dis = jnp.where(d > 0, lax.rsqrt(d), 0.0)
    agg = jnp.concatenate([a0_ref[...], a1_ref[...]], axis=1)
    o_ref[...] = agg * dis[:, None] + b_ref[...]


_tc_out = pl.pallas_call(
    _out_body,
    grid=(N // _RB,),
    in_specs=[
        pl.BlockSpec((_RB, DH), lambda i: (i, 0)),
        pl.BlockSpec((_RB, DH), lambda i: (i, 0)),
        pl.BlockSpec((NC, _RB, 128), lambda i: (0, i, 0)),
        pl.BlockSpec((1, D), lambda i: (0, 0)),
    ],
    out_specs=pl.BlockSpec((_RB, D), lambda i: (i, 0)),
    out_shape=jax.ShapeDtypeStruct((N, D), jnp.float32),
)


def kernel(x, edge_index, W, bias):
    row = edge_index[0]
    col = edge_index[1]
    pad = EPAD - E
    rowp = jnp.pad(row, (0, pad)).reshape(EC, CH)
    colp = jnp.pad(col, (0, pad), constant_values=N).reshape(EC, CH)

    z_deg = jnp.zeros((NPAD, 128), jnp.float32)
    ones_deg = jnp.ones((CH, 128), jnp.float32)
    z_agg = jnp.zeros((NPAD, DH), jnp.float32)

    degp = _sc_deg(colp, z_deg, ones_deg)
    degp_n = degp[:, :N, :]
    h0, h1 = _tc_linear(x, W, degp_n)
    agg0, agg1 = _sc_agg(rowp, colp, h0, h1, z_agg)
    return _tc_out(agg0[:N], agg1[:N], degp_n, bias.reshape(1, D))
